# baseline (speedup 1.0000x reference)
import functools

import jax
import jax.numpy as jnp
import numpy as np
from jax import lax
from jax.experimental import pallas as pl
from jax.experimental.pallas import tpu as pltpu
from jax.experimental.pallas import tpu_sc as plsc

N = 10000
D = 128
E = 320000
E2 = E + N
NC = 2
NS = 16
NW = NC * NS
K = 128
EPC = 10496
NCHUNK = EPC // K
EP = EPC * NW
N2 = 10240
RPT = N2 // NS

_LOOPS = np.arange(N, dtype=np.int32)
_PAD = np.arange(EP - E2, dtype=np.int32) % N


_Z = np.int32(0)
_ONE = np.int32(1)
_RB = 1000
_NB = N // _RB


def _dot(a, b):
    return jnp.dot(a, b, preferred_element_type=jnp.float32,
                   precision=lax.Precision.HIGHEST)


def _front_body(x_ref, w_ref, as_ref, ad_ref, h_ref, av_ref, bv_ref, m_ref, acc):
    i = pl.program_id(0)
    h = _dot(x_ref[...], w_ref[...])
    h_ref[...] = h
    av = jnp.sum(h * as_ref[...], axis=1, keepdims=True)
    bv = jnp.sum(h * ad_ref[...], axis=1, keepdims=True)
    av_ref[...] = av
    bv_ref[...] = bv
    ma, mb = jnp.max(av), jnp.max(bv)

    @pl.when(i == 0)
    def _():
        acc[0] = ma
        acc[1] = mb

    @pl.when(i > 0)
    def _():
        acc[0] = jnp.maximum(acc[0], ma)
        acc[1] = jnp.maximum(acc[1], mb)

    @pl.when(i == pl.num_programs(0) - 1)
    def _():
        s = acc[0] + acc[1]
        m_ref[...] = jnp.full((1, 1), jnp.where(s >= 0, s, 0.2 * s))


def _tc_front(x, W, a_s, a_d):
    return pl.pallas_call(
        _front_body,
        grid=(_NB,),
        in_specs=[
            pl.BlockSpec((_RB, D), lambda i: (i, _Z)),
            pl.BlockSpec((D, D), lambda i: (_Z, _Z)),
            pl.BlockSpec((1, D), lambda i: (_Z, _Z)),
            pl.BlockSpec((1, D), lambda i: (_Z, _Z)),
        ],
        out_specs=[
            pl.BlockSpec((_RB, D), lambda i: (i, _Z)),
            pl.BlockSpec((_RB, 1), lambda i: (i, _Z)),
            pl.BlockSpec((_RB, 1), lambda i: (i, _Z)),
            pl.BlockSpec((1, 1), lambda i: (_Z, _Z)),
        ],
        out_shape=[
            jax.ShapeDtypeStruct((N, D), jnp.float32),
            jax.ShapeDtypeStruct((N, 1), jnp.float32),
            jax.ShapeDtypeStruct((N, 1), jnp.float32),
            jax.ShapeDtypeStruct((1, 1), jnp.float32),
        ],
        scratch_shapes=[pltpu.SMEM((2,), jnp.float32)],
    )(x, W, a_s, a_d)


def _mid_body(p_ref, d_ref, b_ref, w_ref, as_ref, ad_ref,
              h_ref, av_ref, bv_ref, m_ref, acc):
    i = pl.program_id(0)
    num = p_ref[0] + p_ref[1]
    den = d_ref[0] + d_ref[1] + 1e-16
    x1 = jnp.maximum(num / den + b_ref[...], 0.0)
    h = _dot(x1, w_ref[...])
    h_ref[...] = h
    av = jnp.sum(h * as_ref[...], axis=1, keepdims=True)
    bv = jnp.sum(h * ad_ref[...], axis=1, keepdims=True)
    av_ref[...] = av
    bv_ref[...] = bv
    ma, mb = jnp.max(av), jnp.max(bv)

    @pl.when(i == 0)
    def _():
        acc[0] = ma
        acc[1] = mb

    @pl.when(i > 0)
    def _():
        acc[0] = jnp.maximum(acc[0], ma)
        acc[1] = jnp.maximum(acc[1], mb)

    @pl.when(i == pl.num_programs(0) - 1)
    def _():
        s = acc[0] + acc[1]
        m_ref[...] = jnp.full((1, 1), jnp.where(s >= 0, s, 0.2 * s))


def _tc_mid(p, d, b, W, a_s, a_d):
    return pl.pallas_call(
        _mid_body,
        grid=(_NB,),
        in_specs=[
            pl.BlockSpec((NC, _RB, D), lambda i: (_Z, i, _Z)),
            pl.BlockSpec((NC, _RB, 1), lambda i: (_Z, i, _Z)),
            pl.BlockSpec((1, D), lambda i: (_Z, _Z)),
            pl.BlockSpec((D, D), lambda i: (_Z, _Z)),
            pl.BlockSpec((1, D), lambda i: (_Z, _Z)),
            pl.BlockSpec((1, D), lambda i: (_Z, _Z)),
        ],
        out_specs=[
            pl.BlockSpec((_RB, D), lambda i: (i, _Z)),
            pl.BlockSpec((_RB, 1), lambda i: (i, _Z)),
            pl.BlockSpec((_RB, 1), lambda i: (i, _Z)),
            pl.BlockSpec((1, 1), lambda i: (_Z, _Z)),
        ],
        out_shape=[
            jax.ShapeDtypeStruct((N, D), jnp.float32),
            jax.ShapeDtypeStruct((N, 1), jnp.float32),
            jax.ShapeDtypeStruct((N, 1), jnp.float32),
            jax.ShapeDtypeStruct((1, 1), jnp.float32),
        ],
        scratch_shapes=[pltpu.SMEM((2,), jnp.float32)],
    )(p, d, b, W, a_s, a_d)


def _final_body(p_ref, d_ref, b_ref, o_ref):
    num = p_ref[0] + p_ref[1]
    den = d_ref[0] + d_ref[1] + 1e-16
    o_ref[...] = num / den + b_ref[...]


def _tc_final(p, d, b):
    return pl.pallas_call(
        _final_body,
        grid=(_NB,),
        in_specs=[
            pl.BlockSpec((NC, _RB, D), lambda i: (_Z, i, _Z)),
            pl.BlockSpec((NC, _RB, 1), lambda i: (_Z, i, _Z)),
            pl.BlockSpec((1, D), lambda i: (_Z, _Z)),
        ],
        out_specs=pl.BlockSpec((_RB, D), lambda i: (i, _Z)),
        out_shape=jax.ShapeDtypeStruct((N, D), jnp.float32),
    )(p, d, b)


@functools.cache
def _edge_pass_fn():
    mesh = plsc.VectorSubcoreMesh(core_axis_name="c", subcore_axis_name="s",
                                  num_cores=NC, num_subcores=NS)
    return functools.partial(
        pl.kernel,
        out_type=(jax.ShapeDtypeStruct((NC, N2, D), jnp.float32),
                  jax.ShapeDtypeStruct((NC, N2), jnp.float32)),
        mesh=mesh,
        compiler_params=pltpu.CompilerParams(needs_layout_passes=False),
        scratch_types=[
            pltpu.VMEM((16,), jnp.float32),
            pltpu.VMEM((2, K), jnp.float32),
            pltpu.VMEM((2, K), jnp.float32),
            pltpu.VMEM((2, K), jnp.int32),
            pltpu.VMEM((2, K), jnp.int32),
            pltpu.VMEM((K,), jnp.float32),
            pltpu.VMEM((K,), jnp.float32),
            pltpu.VMEM((K, D), jnp.float32),
            pltpu.VMEM((K, D), jnp.float32),
            pltpu.VMEM((640,), jnp.float32),
            pltpu.VMEM((K,), jnp.int32),
            pltpu.VMEM((K,), jnp.int32),
            pltpu.VMEM_SHARED((N2, D), jnp.float32),
            pltpu.VMEM_SHARED((N2,), jnp.float32),
            pltpu.SemaphoreType.DMA,
            pltpu.SemaphoreType.DMA,
            pltpu.SemaphoreType.DMA,
            pltpu.SemaphoreType.DMA,
            pltpu.SemaphoreType.DMA,
            pltpu.SemaphoreType.DMA,
        ],
    )(_edge_pass_body)


def _edge_pass(*args):
    return _edge_pass_fn()(*args)


def _edge_pass_body(h_hbm, sdx_hbm, as_hbm, ad_hbm, m_hbm,
                    outp_hbm, denp_hbm,
                    mv, ab0, ab1, sd0, sd1, wv0, wv1, rows0, rows1, zbuf,
                    di0, di1, out_sh, den_sh, semg0, semg1, semi0, semi1,
                    sems0, sems1):
    i32 = jnp.int32
    c = lax.axis_index("c").astype(i32)
    s = lax.axis_index("s").astype(i32)
    cbase = (c * i32(NS) + s) * i32(NCHUNK)

    pltpu.sync_copy(m_hbm, mv)

    z16 = jnp.zeros((16,), jnp.float32)

    def _zrow(k, carry):
        for g in range(8):
            rows0[k, pl.ds(g * 16, 16)] = z16
        return carry

    lax.fori_loop(jnp.int32(0), jnp.int32(K), _zrow, jnp.int32(0))

    def _zbufi(k, carry):
        zbuf[pl.ds(k * jnp.int32(16), 16)] = z16
        return carry

    lax.fori_loop(jnp.int32(0), jnp.int32(40), _zbufi, jnp.int32(0))

    r0 = s * i32(RPT)
    for q in range(5):
        pltpu.sync_copy(rows0, out_sh.at[pl.ds(r0 + i32(q * 128), 128)])

    pltpu.sync_copy(zbuf, den_sh.at[pl.ds(s * i32(640), 640)])

    plsc.subcore_barrier()

    mvv = mv[...]
    lanes = lax.iota(jnp.int32, 16)

    def _process(j, sd_p, ab_p, wv_p, rows_p, di_p, semg_p, sems_p, sd_q,
                 ab_q, wv_q, rows_q, di_q, semg_q, sems_q, semi_q, semi_p):
        pltpu.make_async_copy(h_hbm.at[sd_p.at[_Z]], rows_p, semg_p).wait()
        pltpu.make_async_copy(as_hbm.at[sd_p.at[_Z]], ab_p.at[_Z], semg_p).wait()
        pltpu.make_async_copy(ad_hbm.at[sd_p.at[_ONE]], ab_p.at[_ONE], semg_p).wait()

        @pl.when(j < i32(NCHUNK - 1))
        def _():
            pltpu.make_async_copy(sdx_hbm.at[cbase + j + i32(1)], sd_q,
                                  semi_q).wait()

            @pl.when(j >= i32(1))
            def _():
                pltpu.make_async_copy(as_hbm.at[pl.ds(_Z, K)], wv_q,
                                      sems_q).wait()
                pltpu.make_async_copy(h_hbm.at[pl.ds(_Z, K)], rows_q,
                                      sems_q).wait()

            pltpu.async_copy(h_hbm.at[sd_q.at[_Z]], rows_q, semg_q)
            pltpu.async_copy(as_hbm.at[sd_q.at[_Z]], ab_q.at[_Z], semg_q)
            pltpu.async_copy(ad_hbm.at[sd_q.at[_ONE]], ab_q.at[_ONE], semg_q)

        eoff = (cbase + j) * i32(K)
        for u in range(8):
            di_p[pl.ds(u * 16, 16)] = sd_p[_ONE, pl.ds(u * 16, 16)]
            e = ab_p[_Z, pl.ds(u * 16, 16)] + ab_p[_ONE, pl.ds(u * 16, 16)]
            e = jnp.where(e >= 0, e, jnp.float32(0.2) * e)
            w = jnp.exp(e - mvv)
            gi = eoff + i32(u * 16) + lanes
            w = jnp.where(gi < i32(E2), w, jnp.float32(0.0))
            wv_p[pl.ds(u * 16, 16)] = w

        @plsc.parallel_loop(jnp.int32(0), jnp.int32(8), jnp.int32(1), unroll=2)
        def _scale(g):
            g16 = g * i32(16)
            w16 = wv_p[pl.ds(g16, 16)]
            for k in range(16):
                rk = g16 + i32(k)
                wb = jnp.full((16,), w16[k])
                for cg in range(8):
                    rows_p[rk, pl.ds(cg * 16, 16)] = (
                        rows_p[rk, pl.ds(cg * 16, 16)] * wb)
        pltpu.async_copy(wv_p, den_sh.at[di_p], sems_p, add=True)
        pltpu.async_copy(rows_p, out_sh.at[di_p], sems_p, add=True)

        @pl.when(j < i32(NCHUNK - 2))
        def _():
            pltpu.async_copy(sdx_hbm.at[cbase + j + i32(2)], sd_p, semi_p)

    pltpu.sync_copy(sdx_hbm.at[cbase], sd0)
    pltpu.async_copy(h_hbm.at[sd0.at[_Z]], rows0, semg0)
    pltpu.async_copy(as_hbm.at[sd0.at[_Z]], ab0.at[_Z], semg0)
    pltpu.async_copy(ad_hbm.at[sd0.at[_ONE]], ab0.at[_ONE], semg0)
    pltpu.async_copy(sdx_hbm.at[cbase + i32(1)], sd1, semi1)

    def _outer(jj, carry):
        j = jj * i32(2)
        _process(j, sd0, ab0, wv0, rows0, di0, semg0, sems0,
                 sd1, ab1, wv1, rows1, di1, semg1, sems1, semi1, semi0)
        _process(j + i32(1), sd1, ab1, wv1, rows1, di1, semg1, sems1,
                 sd0, ab0, wv0, rows0, di0, semg0, sems0, semi0, semi1)
        return carry

    lax.fori_loop(jnp.int32(0), jnp.int32(NCHUNK // 2), _outer, jnp.int32(0))

    pltpu.make_async_copy(as_hbm.at[pl.ds(_Z, K)], wv0, sems0).wait()
    pltpu.make_async_copy(h_hbm.at[pl.ds(_Z, K)], rows0, sems0).wait()
    pltpu.make_async_copy(as_hbm.at[pl.ds(_Z, K)], wv1, sems1).wait()
    pltpu.make_async_copy(h_hbm.at[pl.ds(_Z, K)], rows1, sems1).wait()

    plsc.subcore_barrier()

    for q in range(5):
        off = r0 + i32(q * 128)
        pltpu.sync_copy(out_sh.at[pl.ds(off, 128)], rows0)
        pltpu.sync_copy(rows0, outp_hbm.at[c, pl.ds(off, 128)])

    pltpu.sync_copy(den_sh.at[pl.ds(s * i32(640), 640)], zbuf)
    pltpu.sync_copy(zbuf, denp_hbm.at[c, pl.ds(s * i32(640), 640)])


def kernel(x, edge_index, W1, a1_src, a1_dst, b1, W2, a2_src, a2_dst, b2):
    out_dtype = jnp.result_type(x, W1, b1)
    f = jnp.float32
    x, W1, a1_src, a1_dst, b1 = (a.astype(f) for a in (x, W1, a1_src, a1_dst, b1))
    W2, a2_src, a2_dst, b2 = (a.astype(f) for a in (W2, a2_src, a2_dst, b2))
    loops = jnp.asarray(_LOOPS)
    pad = jnp.asarray(_PAD)
    src = jnp.concatenate([edge_index[0].astype(jnp.int32), loops, pad])
    dst = jnp.concatenate([edge_index[1].astype(jnp.int32), loops, pad])
    sdx = jnp.stack([src.reshape(NW * NCHUNK, K), dst.reshape(NW * NCHUNK, K)],
                    axis=1)

    h1, av1, bv1, m1 = _tc_front(x, W1, a1_src.reshape(1, D), a1_dst.reshape(1, D))
    m1v = jnp.broadcast_to(m1.reshape(1), (16,))
    p1, d1 = _edge_pass(h1, sdx, av1.reshape(N), bv1.reshape(N), m1v)

    h2, av2, bv2, m2 = _tc_mid(p1, d1.reshape(NC, N2, 1), b1.reshape(1, D),
                               W2, a2_src.reshape(1, D), a2_dst.reshape(1, D))
    m2v = jnp.broadcast_to(m2.reshape(1), (16,))
    p2, d2 = _edge_pass(h2, sdx, av2.reshape(N), bv2.reshape(N), m2v)

    return _tc_final(p2, d2.reshape(NC, N2, 1), b2.reshape(1, D)).astype(out_dtype)


# speedup vs baseline: 1.1593x; 1.1593x over previous
import functools

import jax
import jax.numpy as jnp
import numpy as np
from jax import lax
from jax.experimental import pallas as pl
from jax.experimental.pallas import tpu as pltpu
from jax.experimental.pallas import tpu_sc as plsc

N = 10000
D = 128
E = 320000
E2 = E + N
NC = 2
NS = 16
NW = NC * NS
K = 128
EPC = 10496
NCHUNK = EPC // K
EP = EPC * NW
N2 = 10240
RPT = N2 // NS

_LOOPS = np.arange(N, dtype=np.int32)
_PAD = np.arange(EP - E2, dtype=np.int32) % N


_Z = np.int32(0)
_ONE = np.int32(1)
_RB = 1000
_NB = N // _RB


def _dot(a, b):
    return jnp.dot(a, b, preferred_element_type=jnp.float32,
                   precision=lax.Precision.HIGHEST)


def _front_body(x_ref, w_ref, as_ref, ad_ref, h_ref, av_ref, bv_ref, m_ref, acc):
    i = pl.program_id(0)
    h = _dot(x_ref[...], w_ref[...])
    h_ref[...] = h
    av = jnp.sum(h * as_ref[...], axis=1, keepdims=True)
    bv = jnp.sum(h * ad_ref[...], axis=1, keepdims=True)
    av_ref[...] = av
    bv_ref[...] = bv
    ma, mb = jnp.max(av), jnp.max(bv)

    @pl.when(i == 0)
    def _():
        acc[0] = ma
        acc[1] = mb

    @pl.when(i > 0)
    def _():
        acc[0] = jnp.maximum(acc[0], ma)
        acc[1] = jnp.maximum(acc[1], mb)

    @pl.when(i == pl.num_programs(0) - 1)
    def _():
        s = acc[0] + acc[1]
        m_ref[...] = jnp.full((1, 1), jnp.where(s >= 0, s, 0.2 * s))


def _tc_front(x, W, a_s, a_d):
    return pl.pallas_call(
        _front_body,
        grid=(_NB,),
        in_specs=[
            pl.BlockSpec((_RB, D), lambda i: (i, _Z)),
            pl.BlockSpec((D, D), lambda i: (_Z, _Z)),
            pl.BlockSpec((1, D), lambda i: (_Z, _Z)),
            pl.BlockSpec((1, D), lambda i: (_Z, _Z)),
        ],
        out_specs=[
            pl.BlockSpec((_RB, D), lambda i: (i, _Z)),
            pl.BlockSpec((_RB, 1), lambda i: (i, _Z)),
            pl.BlockSpec((_RB, 1), lambda i: (i, _Z)),
            pl.BlockSpec((1, 1), lambda i: (_Z, _Z)),
        ],
        out_shape=[
            jax.ShapeDtypeStruct((N, D), jnp.float32),
            jax.ShapeDtypeStruct((N, 1), jnp.float32),
            jax.ShapeDtypeStruct((N, 1), jnp.float32),
            jax.ShapeDtypeStruct((1, 1), jnp.float32),
        ],
        scratch_shapes=[pltpu.SMEM((2,), jnp.float32)],
    )(x, W, a_s, a_d)


def _mid_body(p_ref, d_ref, b_ref, w_ref, as_ref, ad_ref,
              h_ref, av_ref, bv_ref, m_ref, acc):
    i = pl.program_id(0)
    num = p_ref[0] + p_ref[1]
    den = d_ref[0] + d_ref[1] + 1e-16
    x1 = jnp.maximum(num / den + b_ref[...], 0.0)
    h = _dot(x1, w_ref[...])
    h_ref[...] = h
    av = jnp.sum(h * as_ref[...], axis=1, keepdims=True)
    bv = jnp.sum(h * ad_ref[...], axis=1, keepdims=True)
    av_ref[...] = av
    bv_ref[...] = bv
    ma, mb = jnp.max(av), jnp.max(bv)

    @pl.when(i == 0)
    def _():
        acc[0] = ma
        acc[1] = mb

    @pl.when(i > 0)
    def _():
        acc[0] = jnp.maximum(acc[0], ma)
        acc[1] = jnp.maximum(acc[1], mb)

    @pl.when(i == pl.num_programs(0) - 1)
    def _():
        s = acc[0] + acc[1]
        m_ref[...] = jnp.full((1, 1), jnp.where(s >= 0, s, 0.2 * s))


def _tc_mid(p, d, b, W, a_s, a_d):
    return pl.pallas_call(
        _mid_body,
        grid=(_NB,),
        in_specs=[
            pl.BlockSpec((NC, _RB, D), lambda i: (_Z, i, _Z)),
            pl.BlockSpec((NC, _RB, 1), lambda i: (_Z, i, _Z)),
            pl.BlockSpec((1, D), lambda i: (_Z, _Z)),
            pl.BlockSpec((D, D), lambda i: (_Z, _Z)),
            pl.BlockSpec((1, D), lambda i: (_Z, _Z)),
            pl.BlockSpec((1, D), lambda i: (_Z, _Z)),
        ],
        out_specs=[
            pl.BlockSpec((_RB, D), lambda i: (i, _Z)),
            pl.BlockSpec((_RB, 1), lambda i: (i, _Z)),
            pl.BlockSpec((_RB, 1), lambda i: (i, _Z)),
            pl.BlockSpec((1, 1), lambda i: (_Z, _Z)),
        ],
        out_shape=[
            jax.ShapeDtypeStruct((N, D), jnp.float32),
            jax.ShapeDtypeStruct((N, 1), jnp.float32),
            jax.ShapeDtypeStruct((N, 1), jnp.float32),
            jax.ShapeDtypeStruct((1, 1), jnp.float32),
        ],
        scratch_shapes=[pltpu.SMEM((2,), jnp.float32)],
    )(p, d, b, W, a_s, a_d)


def _final_body(p_ref, d_ref, b_ref, o_ref):
    num = p_ref[0] + p_ref[1]
    den = d_ref[0] + d_ref[1] + 1e-16
    o_ref[...] = num / den + b_ref[...]


def _tc_final(p, d, b):
    return pl.pallas_call(
        _final_body,
        grid=(_NB,),
        in_specs=[
            pl.BlockSpec((NC, _RB, D), lambda i: (_Z, i, _Z)),
            pl.BlockSpec((NC, _RB, 1), lambda i: (_Z, i, _Z)),
            pl.BlockSpec((1, D), lambda i: (_Z, _Z)),
        ],
        out_specs=pl.BlockSpec((_RB, D), lambda i: (i, _Z)),
        out_shape=jax.ShapeDtypeStruct((N, D), jnp.float32),
    )(p, d, b)


@functools.cache
def _edge_pass_fn():
    mesh = plsc.VectorSubcoreMesh(core_axis_name="c", subcore_axis_name="s",
                                  num_cores=NC, num_subcores=NS)
    return functools.partial(
        pl.kernel,
        out_type=(jax.ShapeDtypeStruct((NC, N2, D), jnp.float32),
                  jax.ShapeDtypeStruct((NC, N2), jnp.float32)),
        mesh=mesh,
        compiler_params=pltpu.CompilerParams(needs_layout_passes=False),
        scratch_types=[
            pltpu.VMEM((16,), jnp.float32),
            pltpu.VMEM((N,), jnp.int32),
            pltpu.VMEM((2, K), jnp.int32),
            pltpu.VMEM((2, K), jnp.int32),
            pltpu.VMEM((K,), jnp.float32),
            pltpu.VMEM((K,), jnp.float32),
            pltpu.VMEM((K, D), jnp.float32),
            pltpu.VMEM((K, D), jnp.float32),
            pltpu.VMEM((640,), jnp.float32),
            pltpu.VMEM((K,), jnp.int32),
            pltpu.VMEM((K,), jnp.int32),
            pltpu.VMEM_SHARED((N2, D), jnp.float32),
            pltpu.VMEM_SHARED((N2,), jnp.float32),
            pltpu.SemaphoreType.DMA,
            pltpu.SemaphoreType.DMA,
            pltpu.SemaphoreType.DMA,
            pltpu.SemaphoreType.DMA,
            pltpu.SemaphoreType.DMA,
            pltpu.SemaphoreType.DMA,
        ],
    )(_edge_pass_body)


def _edge_pass(*args):
    return _edge_pass_fn()(*args)


def _edge_pass_body(h_hbm, sdx_hbm, pair_hbm, m_hbm,
                    outp_hbm, denp_hbm,
                    mv, pairs, sd0, sd1, wv0, wv1, rows0, rows1, zbuf,
                    di0, di1, out_sh, den_sh, semg0, semg1, semi0, semi1,
                    sems0, sems1):
    i32 = jnp.int32
    c = lax.axis_index("c").astype(i32)
    s = lax.axis_index("s").astype(i32)
    cbase = (c * i32(NS) + s) * i32(NCHUNK)

    pltpu.sync_copy(m_hbm, mv)
    pltpu.sync_copy(pair_hbm, pairs)

    z16 = jnp.zeros((16,), jnp.float32)

    def _zrow(k, carry):
        for g in range(8):
            rows0[k, pl.ds(g * 16, 16)] = z16
        return carry

    lax.fori_loop(jnp.int32(0), jnp.int32(K), _zrow, jnp.int32(0))

    def _zbufi(k, carry):
        zbuf[pl.ds(k * jnp.int32(16), 16)] = z16
        return carry

    lax.fori_loop(jnp.int32(0), jnp.int32(40), _zbufi, jnp.int32(0))

    r0 = s * i32(RPT)
    for q in range(5):
        pltpu.sync_copy(rows0, out_sh.at[pl.ds(r0 + i32(q * 128), 128)])

    pltpu.sync_copy(zbuf, den_sh.at[pl.ds(s * i32(640), 640)])

    plsc.subcore_barrier()

    mvv = mv[...]
    lanes = lax.iota(jnp.int32, 16)

    def _process(j, sd_p, wv_p, rows_p, di_p, semg_p, sems_p, sd_q,
                 wv_q, rows_q, di_q, semg_q, sems_q, semi_q, semi_p):
        pltpu.make_async_copy(h_hbm.at[sd_p.at[_Z]], rows_p, semg_p).wait()

        @pl.when(j < i32(NCHUNK - 1))
        def _():
            pltpu.make_async_copy(sdx_hbm.at[cbase + j + i32(1)], sd_q,
                                  semi_q).wait()

            @pl.when(j >= i32(1))
            def _():
                pltpu.make_async_copy(h_hbm.at[_Z], wv_q, sems_q).wait()
                pltpu.make_async_copy(h_hbm.at[pl.ds(_Z, K)], rows_q,
                                      sems_q).wait()

            pltpu.async_copy(h_hbm.at[sd_q.at[_Z]], rows_q, semg_q)

        eoff = (cbase + j) * i32(K)
        for u in range(8):
            s16 = sd_p[_Z, pl.ds(u * 16, 16)]
            d16 = sd_p[_ONE, pl.ds(u * 16, 16)]
            di_p[pl.ds(u * 16, 16)] = d16
            g1 = plsc.load_gather(pairs, [s16])
            g2 = plsc.load_gather(pairs, [d16])
            e = (plsc.bitcast(g1 & i32(-65536), jnp.float32) +
                 plsc.bitcast(jnp.left_shift(g2, i32(16)), jnp.float32))
            e = jnp.where(e >= 0, e, jnp.float32(0.2) * e)
            w = jnp.exp(e - mvv)
            gi = eoff + i32(u * 16) + lanes
            w = jnp.where(gi < i32(E2), w, jnp.float32(0.0))
            wv_p[pl.ds(u * 16, 16)] = w

        @plsc.parallel_loop(jnp.int32(0), jnp.int32(8), jnp.int32(1), unroll=2)
        def _scale(g):
            g16 = g * i32(16)
            w16 = wv_p[pl.ds(g16, 16)]
            for k in range(16):
                rk = g16 + i32(k)
                wb = jnp.full((16,), w16[k])
                for cg in range(8):
                    rows_p[rk, pl.ds(cg * 16, 16)] = (
                        rows_p[rk, pl.ds(cg * 16, 16)] * wb)
        pltpu.async_copy(wv_p, den_sh.at[di_p], sems_p, add=True)
        pltpu.async_copy(rows_p, out_sh.at[di_p], sems_p, add=True)

        @pl.when(j < i32(NCHUNK - 2))
        def _():
            pltpu.async_copy(sdx_hbm.at[cbase + j + i32(2)], sd_p, semi_p)

    pltpu.sync_copy(sdx_hbm.at[cbase], sd0)
    pltpu.async_copy(h_hbm.at[sd0.at[_Z]], rows0, semg0)
    pltpu.async_copy(sdx_hbm.at[cbase + i32(1)], sd1, semi1)

    def _outer(jj, carry):
        j = jj * i32(2)
        _process(j, sd0, wv0, rows0, di0, semg0, sems0,
                 sd1, wv1, rows1, di1, semg1, sems1, semi1, semi0)
        _process(j + i32(1), sd1, wv1, rows1, di1, semg1, sems1,
                 sd0, wv0, rows0, di0, semg0, sems0, semi0, semi1)
        return carry

    lax.fori_loop(jnp.int32(0), jnp.int32(NCHUNK // 2), _outer, jnp.int32(0))

    pltpu.make_async_copy(h_hbm.at[_Z], wv0, sems0).wait()
    pltpu.make_async_copy(h_hbm.at[pl.ds(_Z, K)], rows0, sems0).wait()
    pltpu.make_async_copy(h_hbm.at[_Z], wv1, sems1).wait()
    pltpu.make_async_copy(h_hbm.at[pl.ds(_Z, K)], rows1, sems1).wait()

    plsc.subcore_barrier()

    for q in range(5):
        off = r0 + i32(q * 128)
        pltpu.sync_copy(out_sh.at[pl.ds(off, 128)], rows0)
        pltpu.sync_copy(rows0, outp_hbm.at[c, pl.ds(off, 128)])

    pltpu.sync_copy(den_sh.at[pl.ds(s * i32(640), 640)], zbuf)
    pltpu.sync_copy(zbuf, denp_hbm.at[c, pl.ds(s * i32(640), 640)])


def kernel(x, edge_index, W1, a1_src, a1_dst, b1, W2, a2_src, a2_dst, b2):
    out_dtype = jnp.result_type(x, W1, b1)
    f = jnp.float32
    x, W1, a1_src, a1_dst, b1 = (a.astype(f) for a in (x, W1, a1_src, a1_dst, b1))
    W2, a2_src, a2_dst, b2 = (a.astype(f) for a in (W2, a2_src, a2_dst, b2))
    loops = jnp.asarray(_LOOPS)
    pad = jnp.asarray(_PAD)
    src = jnp.concatenate([edge_index[0].astype(jnp.int32), loops, pad])
    dst = jnp.concatenate([edge_index[1].astype(jnp.int32), loops, pad])
    sdx = jnp.stack([src.reshape(NW * NCHUNK, K), dst.reshape(NW * NCHUNK, K)],
                    axis=1)

    h1, av1, bv1, m1 = _tc_front(x, W1, a1_src.reshape(1, D), a1_dst.reshape(1, D))
    m1v = jnp.broadcast_to(m1.reshape(1), (16,))
    pr1 = lax.bitcast_convert_type(
        jnp.stack([bv1.reshape(N).astype(jnp.bfloat16),
                   av1.reshape(N).astype(jnp.bfloat16)], axis=-1), jnp.int32)
    p1, d1 = _edge_pass(h1, sdx, pr1, m1v)

    h2, av2, bv2, m2 = _tc_mid(p1, d1.reshape(NC, N2, 1), b1.reshape(1, D),
                               W2, a2_src.reshape(1, D), a2_dst.reshape(1, D))
    m2v = jnp.broadcast_to(m2.reshape(1), (16,))
    pr2 = lax.bitcast_convert_type(
        jnp.stack([bv2.reshape(N).astype(jnp.bfloat16),
                   av2.reshape(N).astype(jnp.bfloat16)], axis=-1), jnp.int32)
    p2, d2 = _edge_pass(h2, sdx, pr2, m2v)

    return _tc_final(p2, d2.reshape(NC, N2, 1), b2.reshape(1, D)).astype(out_dtype)
